# baseline (speedup 1.0000x reference)
import functools

import jax
import jax.numpy as jnp
from jax import lax
from jax.experimental import pallas as pl
from jax.experimental.pallas import tpu as pltpu
from jax.experimental.pallas import tpu_sc as plsc

N = 10000
K = 16
C = 128
OUTW = 3 * C + 1
NW = 32
NPTS = 320
NPAD = NW * NPTS
NREG = C // 16
G = 4
GK = G * K
NB = 4
NS = 2


def _edge_kernel(pc_hbm, idx_hbm, out_hbm, dum_hbm, idx_all, dsc,
                 nbs, cens, sts, gsems, ssems):
    wid = lax.axis_index("s") * 2 + lax.axis_index("c")
    start = wid * NPTS
    nloc = jnp.minimum(NPTS, N - start)
    nquad = nloc // (NB * G)
    iota = lax.iota(jnp.int32, 16)

    pltpu.sync_copy(idx_hbm.at[pl.ds(start * K, NPTS * K)],
                    idx_all.at[pl.ds(0, NPTS * K)])
    for t in range(NB * GK // 16):
        idx_all[pl.ds(NPTS * K + t * 16, 16)] = jnp.zeros((16,), jnp.int32)

    def gather(c, b):
        pltpu.async_copy(pc_hbm.at[idx_all.at[pl.ds(c * GK, GK)]],
                         nbs[b], gsems[b])
        pltpu.async_copy(pc_hbm.at[pl.ds(start + c * G, G)],
                         cens[b], gsems[b])

    def gather_wait(b):
        pltpu.make_async_copy(pc_hbm.at[idx_all.at[pl.ds(0, GK)]],
                              nbs[b], gsems[b]).wait()
        pltpu.make_async_copy(pc_hbm.at[pl.ds(start, G)],
                              cens[b], gsems[b]).wait()

    def store(c, s):
        pltpu.async_copy(sts[s], out_hbm.at[pl.ds(start * K + c * GK, GK)],
                         ssems[s])

    def store_wait(s):
        pltpu.make_async_copy(sts[s], out_hbm.at[pl.ds(start * K, GK)],
                              ssems[s]).wait()

    def compute(nb, cen, st):
        def pt(g, carry):
            cregs = [cen[g, pl.ds(r * 16, 16)] for r in range(NREG)]
            for j in range(K):
                row = g * K + j
                d = None
                for r in range(NREG):
                    nbr = nb[row, pl.ds(r * 16, 16)]
                    cr = cregs[r]
                    rel = nbr - cr
                    st[row, pl.ds(r * 16, 16)] = cr
                    st[row, pl.ds(C + r * 16, 16)] = nbr
                    st[row, pl.ds(2 * C + r * 16, 16)] = rel
                    sq = rel * rel
                    d = sq if d is None else d + sq
                plsc.store_scatter(dsc, [iota * 16 + j], d)
            dist = dsc[pl.ds(0, 16)]
            for l in range(1, 16):
                dist = dist + dsc[pl.ds(l * 16, 16)]
            plsc.store_scatter(
                st, [g * K + iota, jnp.full((16,), 3 * C, jnp.int32)], dist)
            return carry

        lax.fori_loop(0, G, pt, 0)

    for b in range(NB):
        gather(b, b)
    for s in range(NS):
        pltpu.async_copy(sts[s], dum_hbm.at[wid], ssems[s])

    def quad(cc, carry):
        c0 = NB * cc
        for b in range(NB):
            c = c0 + b
            s = b % NS
            gather_wait(b)
            store_wait(s)
            compute(nbs[b], cens[b], sts[s])
            store(c, s)
            gather(c + NB, b)
        return carry

    lax.fori_loop(0, nquad, quad, 0)

    for b in range(NB):
        gather_wait(b)
    for s in range(NS):
        store_wait(s)


@jax.jit
def kernel(point_cloud, nn_idx):
    pc = point_cloud.reshape(N, C)
    idx = nn_idx.reshape(N * K)
    pc_pad = jnp.pad(pc, ((0, NPAD - N), (0, 0)))
    idx_pad = jnp.pad(idx, (0, (NPAD - N) * K))

    run = functools.partial(
        pl.kernel,
        out_type=(
            jax.ShapeDtypeStruct((N * K, OUTW), jnp.float32),
            jax.ShapeDtypeStruct((NW, GK, OUTW), jnp.float32),
        ),
        mesh=plsc.VectorSubcoreMesh(core_axis_name="c", subcore_axis_name="s"),
        scratch_types=[
            pltpu.VMEM((NPTS * K + NB * GK,), jnp.int32),
            pltpu.VMEM((256,), jnp.float32),
            [pltpu.VMEM((GK, C), jnp.float32)] * NB,
            [pltpu.VMEM((G, C), jnp.float32)] * NB,
            [pltpu.VMEM((GK, OUTW), jnp.float32)] * NS,
            [pltpu.SemaphoreType.DMA] * NB,
            [pltpu.SemaphoreType.DMA] * NS,
        ],
        compiler_params=pltpu.CompilerParams(
            needs_layout_passes=False, use_tc_tiling_on_sc=True),
    )(_edge_kernel)
    out2d, _ = run(pc_pad, idx_pad)
    return out2d.reshape(1, N, K, OUTW)


# speedup vs baseline: 1.1990x; 1.1990x over previous
import functools

import jax
import jax.numpy as jnp
from jax import lax
from jax.experimental import pallas as pl
from jax.experimental.pallas import tpu as pltpu
from jax.experimental.pallas import tpu_sc as plsc

N = 10000
K = 16
C = 128
OUTW = 3 * C + 1
NW = 32
NPTS = 320
NPAD = NW * NPTS
NREG = C // 16
G = 2
GK = G * K


def _edge_kernel(pc_hbm, idx_hbm, out_hbm, dum_hbm, pc_sp, idx_all,
                 nb0, nb1, cen0, cen1, st0, st1, dsc,
                 gsem0, gsem1, ssem0, ssem1):
    wid = lax.axis_index("s") * 2 + lax.axis_index("c")
    start = wid * NPTS
    nloc = jnp.minimum(NPTS, N - start)
    npair = nloc // (2 * G)
    iota = lax.iota(jnp.int32, 16)

    @pl.when(lax.axis_index("s") == 0)
    def _stage_table():
        pltpu.sync_copy(pc_hbm.at[pl.ds(0, N)], pc_sp)

    pltpu.sync_copy(idx_hbm.at[pl.ds(start * K, NPTS * K)],
                    idx_all.at[pl.ds(0, NPTS * K)])
    for t in range(GK // 16):
        idx_all[pl.ds(NPTS * K + t * 16, 16)] = jnp.zeros((16,), jnp.int32)
    plsc.subcore_barrier()

    def gather(c, nb, cen, sem):
        pltpu.async_copy(pc_sp.at[idx_all.at[pl.ds(c * GK, GK)]], nb, sem)
        coff = start + jnp.minimum(c * G, nloc - G)
        pltpu.async_copy(pc_sp.at[pl.ds(coff, G)], cen, sem)

    def gather_wait(nb, cen, sem):
        pltpu.make_async_copy(pc_sp.at[idx_all.at[pl.ds(0, GK)]], nb,
                              sem).wait()
        pltpu.make_async_copy(pc_sp.at[pl.ds(0, G)], cen, sem).wait()

    def store(st, c, sem):
        pltpu.async_copy(st, out_hbm.at[pl.ds(start * K + c * GK, GK)], sem)

    def store_wait(st, sem):
        pltpu.make_async_copy(st, out_hbm.at[pl.ds(start * K, GK)],
                              sem).wait()

    def compute(nb, cen, st):
        def pt(g, carry):
            cregs = [cen[g, pl.ds(r * 16, 16)] for r in range(NREG)]
            for j in range(K):
                row = g * K + j
                d = None
                for r in range(NREG):
                    nbr = nb[row, pl.ds(r * 16, 16)]
                    cr = cregs[r]
                    rel = nbr - cr
                    st[row, pl.ds(r * 16, 16)] = cr
                    st[row, pl.ds(C + r * 16, 16)] = nbr
                    st[row, pl.ds(2 * C + r * 16, 16)] = rel
                    sq = rel * rel
                    d = sq if d is None else d + sq
                plsc.store_scatter(dsc, [iota * 16 + j], d)
            dist = dsc[pl.ds(0, 16)]
            for l in range(1, 16):
                dist = dist + dsc[pl.ds(l * 16, 16)]
            plsc.store_scatter(
                st, [g * K + iota, jnp.full((16,), 3 * C, jnp.int32)], dist)
            return carry

        lax.fori_loop(0, G, pt, 0)

    gather(0, nb0, cen0, gsem0)
    pltpu.async_copy(st0, dum_hbm.at[wid], ssem0)
    pltpu.async_copy(st1, dum_hbm.at[wid], ssem1)

    def pair(cc, carry):
        c0 = 2 * cc
        gather(c0 + 1, nb1, cen1, gsem1)
        gather_wait(nb0, cen0, gsem0)
        store_wait(st0, ssem0)
        compute(nb0, cen0, st0)
        store(st0, c0, ssem0)
        gather(c0 + 2, nb0, cen0, gsem0)
        gather_wait(nb1, cen1, gsem1)
        store_wait(st1, ssem1)
        compute(nb1, cen1, st1)
        store(st1, c0 + 1, ssem1)
        return carry

    lax.fori_loop(0, npair, pair, 0)

    gather_wait(nb0, cen0, gsem0)
    store_wait(st0, ssem0)
    store_wait(st1, ssem1)


@jax.jit
def kernel(point_cloud, nn_idx):
    pc = point_cloud.reshape(N, C)
    idx = nn_idx.reshape(N * K)
    pc_pad = jnp.pad(pc, ((0, NPAD - N), (0, 0)))
    idx_pad = jnp.pad(idx, (0, (NPAD - N) * K))

    run = functools.partial(
        pl.kernel,
        out_type=(
            jax.ShapeDtypeStruct((N * K, OUTW), jnp.float32),
            jax.ShapeDtypeStruct((NW, GK, OUTW), jnp.float32),
        ),
        mesh=plsc.VectorSubcoreMesh(core_axis_name="c", subcore_axis_name="s"),
        scratch_types=[
            pltpu.VMEM_SHARED((N, C), jnp.float32),
            pltpu.VMEM((NPTS * K + GK,), jnp.int32),
            pltpu.VMEM((GK, C), jnp.float32),
            pltpu.VMEM((GK, C), jnp.float32),
            pltpu.VMEM((G, C), jnp.float32),
            pltpu.VMEM((G, C), jnp.float32),
            pltpu.VMEM((GK, OUTW), jnp.float32),
            pltpu.VMEM((GK, OUTW), jnp.float32),
            pltpu.VMEM((256,), jnp.float32),
            pltpu.SemaphoreType.DMA,
            pltpu.SemaphoreType.DMA,
            pltpu.SemaphoreType.DMA,
            pltpu.SemaphoreType.DMA,
        ],
        compiler_params=pltpu.CompilerParams(
            needs_layout_passes=False, use_tc_tiling_on_sc=True),
    )(_edge_kernel)
    out2d, _ = run(pc_pad, idx_pad)
    return out2d.reshape(1, N, K, OUTW)
